# Initial kernel scaffold; baseline (speedup 1.0000x reference)
import jax
import jax.numpy as jnp
from jax.experimental import pallas as pl


def kernel(class_logits, objectness_energy, knownness_energy, pred_boxes, target_sizes):
    raise NotImplementedError("write your pallas kernel here")


# speedup vs baseline: 4.4058x; 4.4058x over previous
import functools

import jax
import jax.numpy as jnp
from jax.experimental import pallas as pl
from jax.experimental.pallas import tpu as pltpu

_INVALID0, _INVALID1 = 100, 102
_C = 128
_QB = 2000
_NB = 10
_K1 = 112
_K = 100
_NEG = -1.0


def _rowmax_kernel(lg_ref, out_ref):
    j = pl.program_id(1)
    lg = lg_ref[0]
    c = jax.lax.broadcasted_iota(jnp.int32, (_QB, _C), 1)
    valid = ((c < _INVALID0) | (c > _INVALID1)) & (c < _C - 1)
    ml = jnp.max(jnp.where(valid, lg, -1e30), axis=1)
    out_ref[0, pl.ds(j, 1), :] = ml[None, :]


def _select_kernel(ml_ref, obj_ref, kno_ref, ts_ref, lg_hbm, bx_hbm,
                   sc_ref, fl_ref, bo_ref,
                   cand_lg, cand_bx, qs_smem, sem1, sem2):
    b = pl.program_id(0)

    ml2 = ml_ref[0]
    op = jnp.clip(jnp.exp(-obj_ref[0]), 1e-6, 1.0)
    kp = jnp.clip(jnp.exp(-kno_ref[0]), 1e-6, 1.0)
    pk2 = op * kp
    pu2 = op * jnp.clip(1.0 - kp, 0.0, 1.0) * 15.0
    m2 = jnp.maximum(pk2 * jax.nn.sigmoid(ml2), pu2)

    flat = (jax.lax.broadcasted_iota(jnp.int32, (_NB, _QB), 0) * _QB
            + jax.lax.broadcasted_iota(jnp.int32, (_NB, _QB), 1))
    onei = jax.lax.broadcasted_iota(jnp.int32, (_K1 + 16, 1), 0)

    def body(i, carry):
        m2, qcol, pkcol, pucol = carry
        mx = jnp.max(m2)
        qstar = jnp.min(jnp.where(m2 == mx, flat, jnp.int32(2**30)))
        qs_smem[i] = qstar
        hit = flat == qstar
        pkv = jnp.sum(jnp.where(hit, pk2, 0.0))
        puv = jnp.sum(jnp.where(hit, pu2, 0.0))
        onec = onei == i
        qcol = qcol + jnp.where(onec, qstar, 0)
        pkcol = pkcol + jnp.where(onec, pkv, 0.0)
        pucol = pucol + jnp.where(onec, puv, 0.0)
        m2 = jnp.where(hit, _NEG, m2)
        return m2, qcol, pkcol, pucol

    qcol0 = jnp.zeros((_K1 + 16, 1), jnp.int32)
    col0 = jnp.zeros((_K1 + 16, 1), jnp.float32)
    _, qcol, pkcol, pucol = jax.lax.fori_loop(
        0, _K1, body, (m2, qcol0, col0, col0))

    copies = []
    for r in range(_K1):
        q = qs_smem[r]
        c1 = pltpu.make_async_copy(lg_hbm.at[b, q], cand_lg.at[r], sem1)
        c1.start()
        c2 = pltpu.make_async_copy(bx_hbm.at[b, q], cand_bx.at[r], sem2)
        c2.start()
        copies.append((c1, c2))
    for c1, c2 in copies:
        c1.wait()
        c2.wait()

    lgc = cand_lg[...]
    c = jax.lax.broadcasted_iota(jnp.int32, (_K1, _C), 1)
    valid = ((c < _INVALID0) | (c > _INVALID1)) & (c < _C - 1)
    pks = jax.lax.slice(pkcol, (0, 0), (_K1, 1))
    pus = jax.lax.slice(pucol, (0, 0), (_K1, 1))
    qss = jax.lax.slice(qcol, (0, 0), (_K1, 1))
    s = jnp.where(valid, pks * jax.nn.sigmoid(lgc), 0.0)
    s = jnp.where(c == _C - 1, pus, s)
    g = qss * _C + c
    sub = jax.lax.broadcasted_iota(jnp.int32, (_K1, _C), 0)
    boxc = cand_bx[...]
    lane1 = jax.lax.broadcasted_iota(jnp.int32, (1, _C), 1)
    sub1 = jax.lax.broadcasted_iota(jnp.int32, (_K1, 1), 0)
    out1 = jax.lax.broadcasted_iota(jnp.int32, (_C, 1), 0)

    def body3(t, carry):
        s, sco, flo, bxo = carry
        mx = jnp.max(s)
        hit = s == mx
        gm = jnp.min(jnp.where(hit, g, jnp.int32(2**30)))
        pick = hit & (g == gm)
        r = jnp.min(jnp.where(pick, sub, jnp.int32(2**30)))
        sco = sco + jnp.where(lane1 == t, mx, 0.0)
        flo = flo + jnp.where(lane1 == t, gm, 0)
        brow = jnp.sum(jnp.where(sub1 == r, boxc, 0.0), axis=0,
                       keepdims=True)
        bxo = bxo + jnp.where(out1 == t, brow, 0.0)
        s = jnp.where(pick, _NEG, s)
        return s, sco, flo, bxo

    sco0 = jnp.zeros((1, _C), jnp.float32)
    flo0 = jnp.zeros((1, _C), jnp.int32)
    bxo0 = jnp.zeros((_C, 4), jnp.float32)
    _, sco, flo, bxo = jax.lax.fori_loop(0, _K, body3, (s, sco0, flo0, bxo0))

    hpx = ts_ref[0, 0, 0].astype(jnp.float32)
    wpx = ts_ref[0, 0, 1].astype(jnp.float32)
    cx = jax.lax.slice(bxo, (0, 0), (_C, 1))
    cy = jax.lax.slice(bxo, (0, 1), (_C, 2))
    w = jax.lax.slice(bxo, (0, 2), (_C, 3))
    h = jax.lax.slice(bxo, (0, 3), (_C, 4))
    xyxy = jnp.concatenate(
        [(cx - 0.5 * w) * wpx, (cy - 0.5 * h) * hpx,
         (cx + 0.5 * w) * wpx, (cy + 0.5 * h) * hpx], axis=1)

    sc_ref[0] = sco
    fl_ref[0] = flo
    bo_ref[0] = xyxy


@jax.jit
def kernel(class_logits, objectness_energy, knownness_energy, pred_boxes,
           target_sizes):
    B, Q, C = class_logits.shape
    assert C == _C and Q == _QB * _NB

    ml = pl.pallas_call(
        _rowmax_kernel,
        grid=(B, _NB),
        in_specs=[pl.BlockSpec((1, _QB, _C), lambda b, j: (b, j, 0))],
        out_specs=pl.BlockSpec((1, _NB, _QB), lambda b, j: (b, 0, 0)),
        out_shape=jax.ShapeDtypeStruct((B, _NB, _QB), jnp.float32),
    )(class_logits)

    obj3 = objectness_energy.reshape(B, _NB, _QB)
    kno3 = knownness_energy.reshape(B, _NB, _QB)
    ts3 = target_sizes.reshape(B, 1, 2)

    grid = (B,)
    sco, flo, bxo = pl.pallas_call(
        _select_kernel,
        grid=grid,
        in_specs=[
            pl.BlockSpec((1, _NB, _QB), lambda b: (b, 0, 0)),
            pl.BlockSpec((1, _NB, _QB), lambda b: (b, 0, 0)),
            pl.BlockSpec((1, _NB, _QB), lambda b: (b, 0, 0)),
            pl.BlockSpec((1, 1, 2), lambda b: (b, 0, 0)),
            pl.BlockSpec(memory_space=pl.ANY),
            pl.BlockSpec(memory_space=pl.ANY),
        ],
        out_specs=[
            pl.BlockSpec((1, 1, _C), lambda b: (b, 0, 0)),
            pl.BlockSpec((1, 1, _C), lambda b: (b, 0, 0)),
            pl.BlockSpec((1, _C, 4), lambda b: (b, 0, 0)),
        ],
        out_shape=[
            jax.ShapeDtypeStruct((B, 1, _C), jnp.float32),
            jax.ShapeDtypeStruct((B, 1, _C), jnp.int32),
            jax.ShapeDtypeStruct((B, _C, 4), jnp.float32),
        ],
        scratch_shapes=[
            pltpu.VMEM((_K1, _C), jnp.float32),
            pltpu.VMEM((_K1, 4), jnp.float32),
            pltpu.SMEM((_K1,), jnp.int32),
            pltpu.SemaphoreType.DMA,
            pltpu.SemaphoreType.DMA,
        ],
    )(ml, obj3, kno3, ts3, class_logits, pred_boxes)

    scores = sco[:, 0, :_K]
    flat = flo[:, 0, :_K]
    labels = flat % _C
    boxes = bxo[:, :_K, :]
    return scores, labels, boxes


# speedup vs baseline: 14.0113x; 3.1802x over previous
import functools

import jax
import jax.numpy as jnp
from jax.experimental import pallas as pl
from jax.experimental.pallas import tpu as pltpu

_INVALID0, _INVALID1 = 100, 102
_C = 128
_QB = 2000
_NB = 10
_B = 8
_K1 = 112
_K = 100
_NEG = -1.0
_BIG = 2**30


def _rowmax_kernel(lg_ref, out_ref):
    j = pl.program_id(1)
    lg = lg_ref[0]
    c = jax.lax.broadcasted_iota(jnp.int32, (_QB, _C), 1)
    valid = ((c < _INVALID0) | (c > _INVALID1)) & (c < _C - 1)
    ml = jnp.max(jnp.where(valid, lg, -1e30), axis=1)
    out_ref[0, pl.ds(j, 1), :] = ml[None, :]


def _select_kernel(ml_ref, obj_ref, kno_ref, ts_ref, lg_hbm, bx_hbm,
                   sc_ref, fl_ref, bo_ref,
                   cand_lg, cand_bx, sem1, sem2):
    ml2 = ml_ref[...]
    op = jnp.clip(jnp.exp(-obj_ref[...]), 1e-6, 1.0)
    kp = jnp.clip(jnp.exp(-kno_ref[...]), 1e-6, 1.0)
    pk2 = op * kp
    pu2 = op * jnp.clip(1.0 - kp, 0.0, 1.0) * 15.0
    m2 = jnp.maximum(pk2 * jax.nn.sigmoid(ml2), pu2)

    flat3 = (jax.lax.broadcasted_iota(jnp.int32, (_B, _NB, _QB), 1) * _QB
             + jax.lax.broadcasted_iota(jnp.int32, (_B, _NB, _QB), 2))
    lane8 = jax.lax.broadcasted_iota(jnp.int32, (_B, _C), 1)

    def body(i, carry):
        m2, qacc, pkacc, puacc = carry
        mx = jnp.max(m2, axis=(1, 2), keepdims=True)
        hit = m2 == mx
        qstar = jnp.min(jnp.where(hit, flat3, _BIG),
                        axis=(1, 2), keepdims=True)
        pick = hit & (flat3 == qstar)
        pkv = jnp.sum(jnp.where(pick, pk2, 0.0), axis=(1, 2), keepdims=True)
        puv = jnp.sum(jnp.where(pick, pu2, 0.0), axis=(1, 2), keepdims=True)
        onec = lane8 == i
        qacc = qacc + jnp.where(onec, qstar[:, :, 0], 0)
        pkacc = pkacc + jnp.where(onec, pkv[:, :, 0], 0.0)
        puacc = puacc + jnp.where(onec, puv[:, :, 0], 0.0)
        for b in range(_B):
            q = qstar[b, 0, 0]
            pltpu.make_async_copy(lg_hbm.at[b, q], cand_lg.at[b, i],
                                  sem1).start()
            pltpu.make_async_copy(bx_hbm.at[b, q], cand_bx.at[b, i],
                                  sem2).start()
        m2 = jnp.where(pick, _NEG, m2)
        return m2, qacc, pkacc, puacc

    qacc0 = jnp.zeros((_B, _C), jnp.int32)
    facc0 = jnp.zeros((_B, _C), jnp.float32)
    _, qacc, pkacc, puacc = jax.lax.fori_loop(
        0, _K1, body, (m2, qacc0, facc0, facc0))

    pltpu.make_async_copy(lg_hbm.at[:, 0:_K1, :], cand_lg, sem1).wait()
    pltpu.make_async_copy(bx_hbm.at[:, 0:_K1, :], cand_bx, sem2).wait()

    lgc = cand_lg[...]
    c = jax.lax.broadcasted_iota(jnp.int32, (_B, _K1, _C), 2)
    valid = ((c < _INVALID0) | (c > _INVALID1)) & (c < _C - 1)
    pks = pkacc[:, :_K1, None]
    pus = puacc[:, :_K1, None]
    qss = qacc[:, :_K1, None]
    s = jnp.where(valid, pks * jax.nn.sigmoid(lgc), 0.0)
    s = jnp.where(c == _C - 1, pus, s)
    g = qss * _C + c
    sub = jax.lax.broadcasted_iota(jnp.int32, (_B, _K1, _C), 1)
    sub_b = jax.lax.broadcasted_iota(jnp.int32, (_B, _K1, 1), 1)
    out_b = jax.lax.broadcasted_iota(jnp.int32, (_B, _C, 1), 1)
    boxc = cand_bx[...]

    def body3(t, carry):
        s, sco, flo, bxo = carry
        mx = jnp.max(s, axis=(1, 2), keepdims=True)
        hit = s == mx
        gm = jnp.min(jnp.where(hit, g, _BIG), axis=(1, 2), keepdims=True)
        pick = hit & (g == gm)
        r = jnp.min(jnp.where(pick, sub, _BIG), axis=(1, 2), keepdims=True)
        onec = lane8 == t
        sco = sco + jnp.where(onec, mx[:, :, 0], 0.0)
        flo = flo + jnp.where(onec, gm[:, :, 0], 0)
        brow = jnp.sum(jnp.where(sub_b == r, boxc, 0.0), axis=1,
                       keepdims=True)
        bxo = bxo + jnp.where(out_b == t, brow, 0.0)
        s = jnp.where(pick, _NEG, s)
        return s, sco, flo, bxo

    sco0 = jnp.zeros((_B, _C), jnp.float32)
    flo0 = jnp.zeros((_B, _C), jnp.int32)
    bxo0 = jnp.zeros((_B, _C, 4), jnp.float32)
    _, sco, flo, bxo = jax.lax.fori_loop(0, _K, body3, (s, sco0, flo0, bxo0))

    ts = ts_ref[...].astype(jnp.float32)
    hpx = ts[:, 0:1, None]
    wpx = ts[:, 1:2, None]
    cx = bxo[:, :, 0:1]
    cy = bxo[:, :, 1:2]
    w = bxo[:, :, 2:3]
    h = bxo[:, :, 3:4]
    xyxy = jnp.concatenate(
        [(cx - 0.5 * w) * wpx, (cy - 0.5 * h) * hpx,
         (cx + 0.5 * w) * wpx, (cy + 0.5 * h) * hpx], axis=2)

    sc_ref[...] = sco
    fl_ref[...] = flo
    bo_ref[...] = xyxy


@jax.jit
def kernel(class_logits, objectness_energy, knownness_energy, pred_boxes,
           target_sizes):
    B, Q, C = class_logits.shape
    assert C == _C and Q == _QB * _NB and B == _B

    ml = pl.pallas_call(
        _rowmax_kernel,
        grid=(B, _NB),
        in_specs=[pl.BlockSpec((1, _QB, _C), lambda b, j: (b, j, 0))],
        out_specs=pl.BlockSpec((1, _NB, _QB), lambda b, j: (b, 0, 0)),
        out_shape=jax.ShapeDtypeStruct((B, _NB, _QB), jnp.float32),
    )(class_logits)

    obj3 = objectness_energy.reshape(B, _NB, _QB)
    kno3 = knownness_energy.reshape(B, _NB, _QB)

    sco, flo, bxo = pl.pallas_call(
        _select_kernel,
        in_specs=[
            pl.BlockSpec((B, _NB, _QB), lambda: (0, 0, 0)),
            pl.BlockSpec((B, _NB, _QB), lambda: (0, 0, 0)),
            pl.BlockSpec((B, _NB, _QB), lambda: (0, 0, 0)),
            pl.BlockSpec((B, 2), lambda: (0, 0)),
            pl.BlockSpec(memory_space=pl.ANY),
            pl.BlockSpec(memory_space=pl.ANY),
        ],
        out_specs=[
            pl.BlockSpec((B, _C), lambda: (0, 0)),
            pl.BlockSpec((B, _C), lambda: (0, 0)),
            pl.BlockSpec((B, _C, 4), lambda: (0, 0, 0)),
        ],
        out_shape=[
            jax.ShapeDtypeStruct((B, _C), jnp.float32),
            jax.ShapeDtypeStruct((B, _C), jnp.int32),
            jax.ShapeDtypeStruct((B, _C, 4), jnp.float32),
        ],
        scratch_shapes=[
            pltpu.VMEM((B, _K1, _C), jnp.float32),
            pltpu.VMEM((B, _K1, 4), jnp.float32),
            pltpu.SemaphoreType.DMA,
            pltpu.SemaphoreType.DMA,
        ],
    )(ml, obj3, kno3, target_sizes, class_logits, pred_boxes)

    scores = sco[:, :_K]
    flat = flo[:, :_K]
    labels = flat % _C
    boxes = bxo[:, :_K, :]
    return scores, labels, boxes
